# baseline (speedup 1.0000x reference)
import functools

import jax
import jax.numpy as jnp
from jax import lax
from jax.experimental import pallas as pl
from jax.experimental.pallas import tpu as pltpu
from jax.experimental.pallas import tpu_sc as plsc

NHEAD = 8
QDIM = 512
TOPK = 8
NQ = 256
SEQ = 2048
INF = 1024
TB = 256
NTB = SEQ // TB
EMB_ROWS = (NQ - 1) * TOPK + (NQ - 1) + 1
EMB_PAD = 2304

_SQRT_2_OVER_PI = 0.7978845608028654


def _top8_packed(s, nbits):
    rows, n = s.shape
    mask = (1 << nbits) - 1
    iota = jax.lax.broadcasted_iota(jnp.int32, (rows, n), 1)
    si = jax.lax.bitcast_convert_type(s, jnp.int32)
    ki = jnp.bitwise_or(jnp.bitwise_and(si, jnp.int32(~mask)), mask - iota)
    key = jax.lax.bitcast_convert_type(ki, jnp.float32)
    vals, idxs = [], []
    for _ in range(TOPK):
        m = jnp.max(key, axis=-1, keepdims=True)
        key = jnp.where(key == m, -jnp.inf, key)
        mb = jax.lax.bitcast_convert_type(m, jnp.int32)
        vals.append(m)
        idxs.append(mask - jnp.bitwise_and(mb, jnp.int32(mask)))
    return jnp.concatenate(vals, axis=1), jnp.concatenate(idxs, axis=1)


def _nt(a, b):
    return jax.lax.dot_general(a, b, (((1,), (1,)), ((), ())),
                               preferred_element_type=jnp.float32)


def _fused_body(inp_f32, inp_bf, wres, wq, wl, wr, wk, embin,
                res_ref, fi_ref, w_ref):
    h = pl.program_id(1)

    x = _nt(inp_bf[...], wq[0])
    xb = x.astype(jnp.bfloat16)
    sl = _nt(xb, wl[...])
    sr = _nt(xb, wr[...])

    lv, li = _top8_packed(sl, 8)
    rv, ri = _top8_packed(sr, 8)

    lrep = jnp.concatenate(
        [jnp.broadcast_to(lv[:, a:a + 1], (TB, TOPK)) for a in range(TOPK)],
        axis=1)
    rtil = jnp.concatenate([rv] * TOPK, axis=1)
    lirep = jnp.concatenate(
        [jnp.broadcast_to(li[:, a:a + 1], (TB, TOPK)) for a in range(TOPK)],
        axis=1)
    ritil = jnp.concatenate([ri] * TOPK, axis=1)
    payload = jnp.bitwise_or(jnp.left_shift(lirep, 8), ritil)
    ci = jnp.bitwise_or(
        jnp.bitwise_and(jax.lax.bitcast_convert_type(lrep + rtil, jnp.int32),
                        jnp.int32(~0xFFFF)), payload)
    ckey = jax.lax.bitcast_convert_type(ci, jnp.float32)

    dots, fibits = [], []
    for _ in range(TOPK):
        m = jnp.max(ckey, axis=-1, keepdims=True)
        ckey = jnp.where(ckey == m, -jnp.inf, ckey)
        dots.append(m)
        fibits.append(jax.lax.bitcast_convert_type(m, jnp.int32))
    dot = jnp.concatenate(dots, axis=1)
    fib = jnp.concatenate(fibits, axis=1)
    fi = (jnp.bitwise_and(jnp.right_shift(fib, 8), 0xFF) * TOPK
          + jnp.bitwise_and(fib, 0xFF))

    e = jnp.exp(dot - jnp.max(dot, axis=-1, keepdims=True))
    scores = e / jnp.sum(e, axis=-1, keepdims=True)

    proj = _nt(inp_bf[...], wk[0])
    ad = _nt(proj.astype(jnp.bfloat16), embin[...])

    lane = jnp.bitwise_and(fi, 127)
    bsel = jnp.right_shift(fi, 7)
    in_dot = jnp.zeros((TB, TOPK), jnp.float32)
    for j in range(EMB_PAD // 128):
        g = jnp.take_along_axis(ad[:, j * 128:(j + 1) * 128], lane, axis=1,
                                mode='promise_in_bounds')
        in_dot = in_dot + jnp.where(bsel == j, g, 0.0)

    g = 0.5 * in_dot * (1.0 + jnp.tanh(
        _SQRT_2_OVER_PI * (in_dot + 0.044715 * in_dot * in_dot * in_dot)))
    w = scores * g

    fi_ref[0] = fi
    w_ref[0] = w

    @pl.when(h == 0)
    def _():
        res_ref[...] = _nt(inp_f32[...], wres[...])


SC_NC = 2
SC_NS = 16
SC_NW = SC_NC * SC_NS
TOK_PER_TILE = SEQ // SC_NW


def _sc_combine(fi_hbm, w_hbm, emb_hbm, res_hbm, out_hbm,
                fi_v, w_v, rows0, rows1, acc_v, sem0, sem1):
    wid = lax.axis_index("s") * SC_NC + lax.axis_index("c")
    ebase = wid * TOK_PER_TILE * TOPK

    for h in range(NHEAD):
        pltpu.sync_copy(fi_hbm.at[h, pl.ds(ebase, TOK_PER_TILE * TOPK)],
                        fi_v.at[pl.ds(h * TOK_PER_TILE * TOPK,
                                      TOK_PER_TILE * TOPK)])
        pltpu.sync_copy(w_hbm.at[h, pl.ds(ebase, TOK_PER_TILE * TOPK)],
                        w_v.at[pl.ds(h * TOK_PER_TILE * TOPK,
                                     TOK_PER_TILE * TOPK)])

    rows = (rows0, rows1)
    sems = (sem0, sem1)
    tbase = wid * TOK_PER_TILE
    hstride = TOK_PER_TILE * TOPK

    def idx_ref(h, t):
        return fi_v.at[pl.ds(h * hstride + t * TOPK, TOPK)]

    pltpu.async_copy(emb_hbm.at[idx_ref(0, 0)], rows0, sem0)
    pltpu.async_copy(emb_hbm.at[idx_ref(1, 0)], rows1, sem1)

    def token_body(t, carry):
        pltpu.sync_copy(res_hbm.at[tbase + t], acc_v)
        for h in range(NHEAD):
            b = h % 2
            pltpu.make_async_copy(emb_hbm.at[idx_ref(0, 0)], rows[b],
                                  sems[b]).wait()
            w8 = w_v[pl.ds(h * hstride + t * TOPK, 16)]
            wks = [lax.gather(
                       w8, jnp.full((16, 1), k, jnp.int32),
                       lax.GatherDimensionNumbers(
                           offset_dims=(), collapsed_slice_dims=(0,),
                           start_index_map=(0,)),
                       (1,), mode=lax.GatherScatterMode.PROMISE_IN_BOUNDS)
                   for k in range(TOPK)]

            def chunk_body(ci, c2, _b=b, _wks=wks):
                off = ci * 16
                a = acc_v[pl.ds(off, 16)]
                for k in range(TOPK):
                    a = a + _wks[k] * rows[_b][k, pl.ds(off, 16)]
                acc_v[pl.ds(off, 16)] = a
                return c2

            lax.fori_loop(0, INF // 16, chunk_body, 0)

            nh = h + 2
            if nh < NHEAD:
                pltpu.async_copy(emb_hbm.at[idx_ref(nh, t)], rows[b], sems[b])
            else:
                @pl.when(t + 1 < TOK_PER_TILE)
                def _():
                    pltpu.async_copy(emb_hbm.at[idx_ref(nh - NHEAD, t + 1)],
                                     rows[b], sems[b])
        pltpu.sync_copy(acc_v, out_hbm.at[tbase + t])
        return carry

    lax.fori_loop(0, TOK_PER_TILE, token_body, 0)


def kernel(inp, W_res, W_q, W_k, W_left, W_right, emb_in, emb_out):
    inp2d = inp.reshape(SEQ, INF)
    inp_bf = inp2d.astype(jnp.bfloat16)
    wq = W_q.reshape(NHEAD, QDIM, INF).astype(jnp.bfloat16)
    wk = W_k.reshape(NHEAD, INF, INF).astype(jnp.bfloat16)
    wl = W_left.astype(jnp.bfloat16)
    wr = W_right.astype(jnp.bfloat16)
    pad = EMB_PAD - EMB_ROWS
    embin = jnp.pad(emb_in[:EMB_ROWS].astype(jnp.bfloat16), ((0, pad), (0, 0)))

    grid = (NTB, NHEAD)
    res, fi3, w3 = pl.pallas_call(
        _fused_body,
        grid=grid,
        in_specs=[
            pl.BlockSpec((TB, INF), lambda tb, h: (tb, 0)),
            pl.BlockSpec((TB, INF), lambda tb, h: (tb, 0)),
            pl.BlockSpec((INF, INF), lambda tb, h: (0, 0)),
            pl.BlockSpec((1, QDIM, INF), lambda tb, h: (h, 0, 0)),
            pl.BlockSpec((NQ, QDIM), lambda tb, h: (0, 0)),
            pl.BlockSpec((NQ, QDIM), lambda tb, h: (0, 0)),
            pl.BlockSpec((1, INF, INF), lambda tb, h: (h, 0, 0)),
            pl.BlockSpec((EMB_PAD, INF), lambda tb, h: (0, 0)),
        ],
        out_specs=[
            pl.BlockSpec((TB, INF), lambda tb, h: (tb, 0)),
            pl.BlockSpec((1, TB, TOPK), lambda tb, h: (h, tb, 0)),
            pl.BlockSpec((1, TB, TOPK), lambda tb, h: (h, tb, 0)),
        ],
        out_shape=[
            jax.ShapeDtypeStruct((SEQ, INF), jnp.float32),
            jax.ShapeDtypeStruct((NHEAD, SEQ, TOPK), jnp.int32),
            jax.ShapeDtypeStruct((NHEAD, SEQ, TOPK), jnp.float32),
        ],
        compiler_params=pltpu.CompilerParams(
            dimension_semantics=("arbitrary", "arbitrary")),
    )(inp2d, inp_bf, W_res, wq, wl, wr, wk, embin)

    sc_fn = pl.kernel(
        _sc_combine,
        mesh=plsc.VectorSubcoreMesh(core_axis_name="c", subcore_axis_name="s"),
        out_type=jax.ShapeDtypeStruct((SEQ, INF), jnp.float32),
        scratch_types=[
            pltpu.VMEM((NHEAD * TOK_PER_TILE * TOPK,), jnp.int32),
            pltpu.VMEM((NHEAD * TOK_PER_TILE * TOPK + 16,), jnp.float32),
            pltpu.VMEM((TOPK, INF), jnp.float32),
            pltpu.VMEM((TOPK, INF), jnp.float32),
            pltpu.VMEM((INF,), jnp.float32),
            pltpu.SemaphoreType.DMA,
            pltpu.SemaphoreType.DMA,
        ],
    )
    out = sc_fn(fi3.reshape(NHEAD, SEQ * TOPK), w3.reshape(NHEAD, SEQ * TOPK),
                emb_out, res)
    return out.reshape(1, SEQ, INF)


# speedup vs baseline: 1.0430x; 1.0430x over previous
import functools

import jax
import jax.numpy as jnp
from jax import lax
from jax.experimental import pallas as pl
from jax.experimental.pallas import tpu as pltpu
from jax.experimental.pallas import tpu_sc as plsc

NHEAD = 8
QDIM = 512
TOPK = 8
NQ = 256
SEQ = 2048
INF = 1024
TB = 256
NTB = SEQ // TB
EMB_ROWS = (NQ - 1) * TOPK + (NQ - 1) + 1
EMB_PAD = 2304

_SQRT_2_OVER_PI = 0.7978845608028654


def _top8_packed(s, nbits):
    rows, n = s.shape
    mask = (1 << nbits) - 1
    iota = jax.lax.broadcasted_iota(jnp.int32, (rows, n), 1)
    si = jax.lax.bitcast_convert_type(s, jnp.int32)
    ki = jnp.bitwise_or(jnp.bitwise_and(si, jnp.int32(~mask)), mask - iota)
    key = jax.lax.bitcast_convert_type(ki, jnp.float32)
    vals, idxs = [], []
    for _ in range(TOPK):
        m = jnp.max(key, axis=-1, keepdims=True)
        key = jnp.where(key == m, -jnp.inf, key)
        mb = jax.lax.bitcast_convert_type(m, jnp.int32)
        vals.append(m)
        idxs.append(mask - jnp.bitwise_and(mb, jnp.int32(mask)))
    return jnp.concatenate(vals, axis=1), jnp.concatenate(idxs, axis=1)


def _nt(a, b):
    return jax.lax.dot_general(a, b, (((1,), (1,)), ((), ())),
                               preferred_element_type=jnp.float32)


def _fused_body(inp_f32, inp_bf, wres, wq, wl, wr, wk, embin,
                res_ref, fi_ref, w_ref):
    h = pl.program_id(1)

    x = _nt(inp_bf[...], wq[0])
    xb = x.astype(jnp.bfloat16)
    sl = _nt(xb, wl[...])
    sr = _nt(xb, wr[...])

    lv, li = _top8_packed(sl, 8)
    rv, ri = _top8_packed(sr, 8)

    lrep = jnp.concatenate(
        [jnp.broadcast_to(lv[:, a:a + 1], (TB, TOPK)) for a in range(TOPK)],
        axis=1)
    rtil = jnp.concatenate([rv] * TOPK, axis=1)
    lirep = jnp.concatenate(
        [jnp.broadcast_to(li[:, a:a + 1], (TB, TOPK)) for a in range(TOPK)],
        axis=1)
    ritil = jnp.concatenate([ri] * TOPK, axis=1)
    payload = jnp.bitwise_or(jnp.left_shift(lirep, 8), ritil)
    ci = jnp.bitwise_or(
        jnp.bitwise_and(jax.lax.bitcast_convert_type(lrep + rtil, jnp.int32),
                        jnp.int32(~0xFFFF)), payload)
    ckey = jax.lax.bitcast_convert_type(ci, jnp.float32)

    dots, fibits = [], []
    for _ in range(TOPK):
        m = jnp.max(ckey, axis=-1, keepdims=True)
        ckey = jnp.where(ckey == m, -jnp.inf, ckey)
        dots.append(m)
        fibits.append(jax.lax.bitcast_convert_type(m, jnp.int32))
    dot = jnp.concatenate(dots, axis=1)
    fib = jnp.concatenate(fibits, axis=1)
    fi = (jnp.bitwise_and(jnp.right_shift(fib, 8), 0xFF) * TOPK
          + jnp.bitwise_and(fib, 0xFF))

    e = jnp.exp(dot - jnp.max(dot, axis=-1, keepdims=True))
    scores = e / jnp.sum(e, axis=-1, keepdims=True)

    proj = _nt(inp_bf[...], wk[0])
    ad = _nt(proj.astype(jnp.bfloat16), embin[...])

    lane = jnp.bitwise_and(fi, 127)
    bsel = jnp.right_shift(fi, 7)
    in_dot = jnp.zeros((TB, TOPK), jnp.float32)
    for j in range(EMB_PAD // 128):
        g = jnp.take_along_axis(ad[:, j * 128:(j + 1) * 128], lane, axis=1,
                                mode='promise_in_bounds')
        in_dot = in_dot + jnp.where(bsel == j, g, 0.0)

    g = 0.5 * in_dot * (1.0 + jnp.tanh(
        _SQRT_2_OVER_PI * (in_dot + 0.044715 * in_dot * in_dot * in_dot)))
    w = scores * g

    fi_ref[0] = fi
    w_ref[0] = w

    @pl.when(h == 0)
    def _():
        res_ref[...] = _nt(inp_f32[...], wres[...])


SC_NC = 2
SC_NS = 16
SC_NW = SC_NC * SC_NS
TOK_PER_TILE = SEQ // SC_NW


def _sc_combine(fi_hbm, w_hbm, emb_hbm, out_hbm,
                fi_v, w_v, rows0, rows1, eacc, sem0, sem1):
    wid = lax.axis_index("s") * SC_NC + lax.axis_index("c")
    ebase = wid * TOK_PER_TILE * TOPK

    for h in range(NHEAD):
        pltpu.sync_copy(fi_hbm.at[pl.ds(h * SEQ * TOPK + ebase,
                                        TOK_PER_TILE * TOPK)],
                        fi_v.at[pl.ds(h * TOK_PER_TILE * TOPK,
                                      TOK_PER_TILE * TOPK)])
        pltpu.sync_copy(w_hbm.at[pl.ds(h * SEQ * TOPK + ebase,
                                       TOK_PER_TILE * TOPK)],
                        w_v.at[pl.ds(h * TOK_PER_TILE * TOPK,
                                     TOK_PER_TILE * TOPK)])

    rows = (rows0, rows1)
    sems = (sem0, sem1)
    tbase = wid * TOK_PER_TILE
    hstride = TOK_PER_TILE * TOPK

    def idx_ref(h, t):
        return fi_v.at[pl.ds(h * hstride + t * TOPK, TOPK)]

    pltpu.async_copy(emb_hbm.at[idx_ref(0, 0)], rows0, sem0)
    pltpu.async_copy(emb_hbm.at[idx_ref(1, 0)], rows1, sem1)

    def token_body(t, carry):
        for h in range(NHEAD):
            b = h % 2
            pltpu.make_async_copy(emb_hbm.at[idx_ref(0, 0)], rows[b],
                                  sems[b]).wait()
            w8 = w_v[pl.ds(h * hstride + t * TOPK, 16)]
            wk16 = [lax.gather(
                        w8, jnp.full((16, 1), k, jnp.int32),
                        lax.GatherDimensionNumbers(
                            offset_dims=(), collapsed_slice_dims=(0,),
                            start_index_map=(0,)),
                        (1,), mode=lax.GatherScatterMode.PROMISE_IN_BOUNDS)
                    for k in range(TOPK)]

            def chunk_body(ci, c2, _b=b, _wk=wk16, _h=h):
                for j in range(4):
                    off = ci * 64 + j * 16
                    p = [_wk[k] * rows[_b][k, pl.ds(off, 16)]
                         for k in range(TOPK)]
                    s = (((p[0] + p[1]) + (p[2] + p[3]))
                         + ((p[4] + p[5]) + (p[6] + p[7])))
                    if _h == 0:
                        eacc[pl.ds(off, 16)] = s
                    else:
                        eacc[pl.ds(off, 16)] = eacc[pl.ds(off, 16)] + s
                return c2

            lax.fori_loop(0, INF // 64, chunk_body, 0)

            nh = h + 2
            if nh < NHEAD:
                pltpu.async_copy(emb_hbm.at[idx_ref(nh, t)], rows[b], sems[b])
            else:
                @pl.when(t + 1 < TOK_PER_TILE)
                def _():
                    pltpu.async_copy(emb_hbm.at[idx_ref(nh - NHEAD, t + 1)],
                                     rows[b], sems[b])
        pltpu.sync_copy(eacc, out_hbm.at[pl.ds((tbase + t) * INF, INF)])
        return carry

    lax.fori_loop(0, TOK_PER_TILE, token_body, 0)


def _add_body(res, eb, out_ref):
    out_ref[...] = res[...] + eb[...].astype(jnp.float32)


def kernel(inp, W_res, W_q, W_k, W_left, W_right, emb_in, emb_out):
    inp2d = inp.reshape(SEQ, INF)
    inp_bf = inp2d.astype(jnp.bfloat16)
    wq = W_q.reshape(NHEAD, QDIM, INF).astype(jnp.bfloat16)
    wk = W_k.reshape(NHEAD, INF, INF).astype(jnp.bfloat16)
    wl = W_left.astype(jnp.bfloat16)
    wr = W_right.astype(jnp.bfloat16)
    pad = EMB_PAD - EMB_ROWS
    embin = jnp.pad(emb_in[:EMB_ROWS].astype(jnp.bfloat16), ((0, pad), (0, 0)))

    grid = (NTB, NHEAD)
    res, fi3, w3 = pl.pallas_call(
        _fused_body,
        grid=grid,
        in_specs=[
            pl.BlockSpec((TB, INF), lambda tb, h: (tb, 0)),
            pl.BlockSpec((TB, INF), lambda tb, h: (tb, 0)),
            pl.BlockSpec((INF, INF), lambda tb, h: (0, 0)),
            pl.BlockSpec((1, QDIM, INF), lambda tb, h: (h, 0, 0)),
            pl.BlockSpec((NQ, QDIM), lambda tb, h: (0, 0)),
            pl.BlockSpec((NQ, QDIM), lambda tb, h: (0, 0)),
            pl.BlockSpec((1, INF, INF), lambda tb, h: (h, 0, 0)),
            pl.BlockSpec((EMB_PAD, INF), lambda tb, h: (0, 0)),
        ],
        out_specs=[
            pl.BlockSpec((TB, INF), lambda tb, h: (tb, 0)),
            pl.BlockSpec((1, TB, TOPK), lambda tb, h: (h, tb, 0)),
            pl.BlockSpec((1, TB, TOPK), lambda tb, h: (h, tb, 0)),
        ],
        out_shape=[
            jax.ShapeDtypeStruct((SEQ, INF), jnp.float32),
            jax.ShapeDtypeStruct((NHEAD, SEQ, TOPK), jnp.int32),
            jax.ShapeDtypeStruct((NHEAD, SEQ, TOPK), jnp.float32),
        ],
        compiler_params=pltpu.CompilerParams(
            dimension_semantics=("arbitrary", "arbitrary")),
    )(inp2d, inp_bf, W_res, wq, wl, wr, wk, embin)

    embout = emb_out[:EMB_PAD]

    sc_fn = pl.kernel(
        _sc_combine,
        mesh=plsc.VectorSubcoreMesh(core_axis_name="c", subcore_axis_name="s"),
        out_type=jax.ShapeDtypeStruct((SEQ * INF,), jnp.float32),
        scratch_types=[
            pltpu.VMEM((NHEAD * TOK_PER_TILE * TOPK,), jnp.int32),
            pltpu.VMEM((NHEAD * TOK_PER_TILE * TOPK + 16,), jnp.float32),
            pltpu.VMEM((TOPK, INF), jnp.float32),
            pltpu.VMEM((TOPK, INF), jnp.float32),
            pltpu.VMEM((INF,), jnp.float32),
            pltpu.SemaphoreType.DMA,
            pltpu.SemaphoreType.DMA,
        ],
    )
    eb = sc_fn(fi3.reshape(-1), w3.reshape(-1), embout).reshape(SEQ, INF)

    out = pl.pallas_call(
        _add_body,
        grid=(NTB,),
        in_specs=[
            pl.BlockSpec((TB, INF), lambda tb: (tb, 0)),
            pl.BlockSpec((TB, INF), lambda tb: (tb, 0)),
        ],
        out_specs=pl.BlockSpec((TB, INF), lambda tb: (tb, 0)),
        out_shape=jax.ShapeDtypeStruct((SEQ, INF), jnp.float32),
    )(res, eb)
    return out.reshape(1, SEQ, INF)


# speedup vs baseline: 1.0661x; 1.0222x over previous
import functools

import jax
import jax.numpy as jnp
from jax import lax
from jax.experimental import pallas as pl
from jax.experimental.pallas import tpu as pltpu
from jax.experimental.pallas import tpu_sc as plsc

NHEAD = 8
QDIM = 512
TOPK = 8
NQ = 256
SEQ = 2048
INF = 1024
TB = 256
NTB = SEQ // TB
EMB_ROWS = (NQ - 1) * TOPK + (NQ - 1) + 1
EMB_PAD = 2304

_SQRT_2_OVER_PI = 0.7978845608028654


def _top8_packed(s, nbits):
    rows, n = s.shape
    mask = (1 << nbits) - 1
    iota = jax.lax.broadcasted_iota(jnp.int32, (rows, n), 1)
    si = jax.lax.bitcast_convert_type(s, jnp.int32)
    ki = jnp.bitwise_or(jnp.bitwise_and(si, jnp.int32(~mask)), mask - iota)
    key = jax.lax.bitcast_convert_type(ki, jnp.float32)
    vals, idxs = [], []
    for _ in range(TOPK):
        m = jnp.max(key, axis=-1, keepdims=True)
        key = jnp.where(key == m, -jnp.inf, key)
        mb = jax.lax.bitcast_convert_type(m, jnp.int32)
        vals.append(m)
        idxs.append(mask - jnp.bitwise_and(mb, jnp.int32(mask)))
    return jnp.concatenate(vals, axis=1), jnp.concatenate(idxs, axis=1)


def _nt(a, b):
    return jax.lax.dot_general(a, b, (((1,), (1,)), ((), ())),
                               preferred_element_type=jnp.float32)


def _fused_body(inp_f32, inp_bf, wres, wq, wl, wr, wk, embin,
                res_ref, fi_ref, w_ref):
    h = pl.program_id(1)

    x = _nt(inp_bf[...], wq[0])
    xb = x.astype(jnp.bfloat16)
    sl = _nt(xb, wl[...])
    sr = _nt(xb, wr[...])

    lv, li = _top8_packed(sl, 8)
    rv, ri = _top8_packed(sr, 8)

    lrep = jnp.concatenate(
        [jnp.broadcast_to(lv[:, a:a + 1], (TB, TOPK)) for a in range(TOPK)],
        axis=1)
    rtil = jnp.concatenate([rv] * TOPK, axis=1)
    lirep = jnp.concatenate(
        [jnp.broadcast_to(li[:, a:a + 1], (TB, TOPK)) for a in range(TOPK)],
        axis=1)
    ritil = jnp.concatenate([ri] * TOPK, axis=1)
    payload = jnp.bitwise_or(jnp.left_shift(lirep, 8), ritil)
    ci = jnp.bitwise_or(
        jnp.bitwise_and(jax.lax.bitcast_convert_type(lrep + rtil, jnp.int32),
                        jnp.int32(~0xFFFF)), payload)
    ckey = jax.lax.bitcast_convert_type(ci, jnp.float32)

    dots, fibits = [], []
    for _ in range(TOPK):
        m = jnp.max(ckey, axis=-1, keepdims=True)
        ckey = jnp.where(ckey == m, -jnp.inf, ckey)
        dots.append(m)
        fibits.append(jax.lax.bitcast_convert_type(m, jnp.int32))
    dot = jnp.concatenate(dots, axis=1)
    fib = jnp.concatenate(fibits, axis=1)
    fi = (jnp.bitwise_and(jnp.right_shift(fib, 8), 0xFF) * TOPK
          + jnp.bitwise_and(fib, 0xFF))

    e = jnp.exp(dot - jnp.max(dot, axis=-1, keepdims=True))
    scores = e / jnp.sum(e, axis=-1, keepdims=True)

    proj = _nt(inp_bf[...], wk[0])
    ad = _nt(proj.astype(jnp.bfloat16), embin[...])

    lane = jnp.bitwise_and(fi, 127)
    bsel = jnp.right_shift(fi, 7)
    in_dot = jnp.zeros((TB, TOPK), jnp.float32)
    for j in range(EMB_PAD // 128):
        g = jnp.take_along_axis(ad[:, j * 128:(j + 1) * 128], lane, axis=1,
                                mode='promise_in_bounds')
        in_dot = in_dot + jnp.where(bsel == j, g, 0.0)

    g = 0.5 * in_dot * (1.0 + jnp.tanh(
        _SQRT_2_OVER_PI * (in_dot + 0.044715 * in_dot * in_dot * in_dot)))
    w = scores * g

    fi_ref[0] = fi
    w_ref[0] = w

    @pl.when(h == 0)
    def _():
        res_ref[...] = _nt(inp_f32[...], wres[...])


SC_NC = 2
SC_NS = 16
SC_NW = SC_NC * SC_NS
TOK_PER_TILE = SEQ // SC_NW


def _sc_combine(fi_hbm, w_hbm, emb_hbm, out_hbm,
                fi_v, w_v, rows0, rows1, rows2, rows3, eacc,
                sem0, sem1, sem2, sem3):
    wid = lax.axis_index("s") * SC_NC + lax.axis_index("c")
    ebase = wid * TOK_PER_TILE * TOPK

    for h in range(NHEAD):
        pltpu.sync_copy(fi_hbm.at[pl.ds(h * SEQ * TOPK + ebase,
                                        TOK_PER_TILE * TOPK)],
                        fi_v.at[pl.ds(h * TOK_PER_TILE * TOPK,
                                      TOK_PER_TILE * TOPK)])
        pltpu.sync_copy(w_hbm.at[pl.ds(h * SEQ * TOPK + ebase,
                                       TOK_PER_TILE * TOPK)],
                        w_v.at[pl.ds(h * TOK_PER_TILE * TOPK,
                                     TOK_PER_TILE * TOPK)])

    rows = (rows0, rows1, rows2, rows3)
    sems = (sem0, sem1, sem2, sem3)
    tbase = wid * TOK_PER_TILE
    hstride = TOK_PER_TILE * TOPK
    NBUF = 4

    def idx_ref(h, t):
        return fi_v.at[pl.ds(h * hstride + t * TOPK, TOPK)]

    for h in range(NBUF):
        pltpu.async_copy(emb_hbm.at[idx_ref(h, 0)], rows[h], sems[h])

    def token_body(t, carry):
        for h in range(NHEAD):
            b = h % NBUF
            pltpu.make_async_copy(emb_hbm.at[idx_ref(0, 0)], rows[b],
                                  sems[b]).wait()
            w8 = w_v[pl.ds(h * hstride + t * TOPK, 16)]
            wk16 = [lax.gather(
                        w8, jnp.full((16, 1), k, jnp.int32),
                        lax.GatherDimensionNumbers(
                            offset_dims=(), collapsed_slice_dims=(0,),
                            start_index_map=(0,)),
                        (1,), mode=lax.GatherScatterMode.PROMISE_IN_BOUNDS)
                    for k in range(TOPK)]

            def chunk_body(ci, c2, _b=b, _wk=wk16, _h=h):
                for j in range(4):
                    off = ci * 64 + j * 16
                    p = [_wk[k] * rows[_b][k, pl.ds(off, 16)]
                         for k in range(TOPK)]
                    s = (((p[0] + p[1]) + (p[2] + p[3]))
                         + ((p[4] + p[5]) + (p[6] + p[7])))
                    if _h == 0:
                        eacc[pl.ds(off, 16)] = s
                    else:
                        eacc[pl.ds(off, 16)] = eacc[pl.ds(off, 16)] + s
                return c2

            lax.fori_loop(0, INF // 64, chunk_body, 0)

            nh = h + NBUF
            if nh < NHEAD:
                pltpu.async_copy(emb_hbm.at[idx_ref(nh, t)], rows[b], sems[b])
            else:
                @pl.when(t + 1 < TOK_PER_TILE)
                def _():
                    pltpu.async_copy(emb_hbm.at[idx_ref(nh - NHEAD, t + 1)],
                                     rows[b], sems[b])
        pltpu.sync_copy(eacc, out_hbm.at[pl.ds((tbase + t) * INF, INF)])
        return carry

    lax.fori_loop(0, TOK_PER_TILE, token_body, 0)


def _add_body(res, eb, out_ref):
    out_ref[...] = res[...] + eb[...].astype(jnp.float32)


def kernel(inp, W_res, W_q, W_k, W_left, W_right, emb_in, emb_out):
    inp2d = inp.reshape(SEQ, INF)
    inp_bf = inp2d.astype(jnp.bfloat16)
    wq = W_q.reshape(NHEAD, QDIM, INF).astype(jnp.bfloat16)
    wk = W_k.reshape(NHEAD, INF, INF).astype(jnp.bfloat16)
    wl = W_left.astype(jnp.bfloat16)
    wr = W_right.astype(jnp.bfloat16)
    pad = EMB_PAD - EMB_ROWS
    embin = jnp.pad(emb_in[:EMB_ROWS].astype(jnp.bfloat16), ((0, pad), (0, 0)))

    grid = (NTB, NHEAD)
    res, fi3, w3 = pl.pallas_call(
        _fused_body,
        grid=grid,
        in_specs=[
            pl.BlockSpec((TB, INF), lambda tb, h: (tb, 0)),
            pl.BlockSpec((TB, INF), lambda tb, h: (tb, 0)),
            pl.BlockSpec((INF, INF), lambda tb, h: (0, 0)),
            pl.BlockSpec((1, QDIM, INF), lambda tb, h: (h, 0, 0)),
            pl.BlockSpec((NQ, QDIM), lambda tb, h: (0, 0)),
            pl.BlockSpec((NQ, QDIM), lambda tb, h: (0, 0)),
            pl.BlockSpec((1, INF, INF), lambda tb, h: (h, 0, 0)),
            pl.BlockSpec((EMB_PAD, INF), lambda tb, h: (0, 0)),
        ],
        out_specs=[
            pl.BlockSpec((TB, INF), lambda tb, h: (tb, 0)),
            pl.BlockSpec((1, TB, TOPK), lambda tb, h: (h, tb, 0)),
            pl.BlockSpec((1, TB, TOPK), lambda tb, h: (h, tb, 0)),
        ],
        out_shape=[
            jax.ShapeDtypeStruct((SEQ, INF), jnp.float32),
            jax.ShapeDtypeStruct((NHEAD, SEQ, TOPK), jnp.int32),
            jax.ShapeDtypeStruct((NHEAD, SEQ, TOPK), jnp.float32),
        ],
        compiler_params=pltpu.CompilerParams(
            dimension_semantics=("arbitrary", "arbitrary")),
    )(inp2d, inp_bf, W_res, wq, wl, wr, wk, embin)

    embout = emb_out[:EMB_PAD]

    sc_fn = pl.kernel(
        _sc_combine,
        mesh=plsc.VectorSubcoreMesh(core_axis_name="c", subcore_axis_name="s"),
        out_type=jax.ShapeDtypeStruct((SEQ * INF,), jnp.float32),
        scratch_types=[
            pltpu.VMEM((NHEAD * TOK_PER_TILE * TOPK,), jnp.int32),
            pltpu.VMEM((NHEAD * TOK_PER_TILE * TOPK + 16,), jnp.float32),
            pltpu.VMEM((TOPK, INF), jnp.float32),
            pltpu.VMEM((TOPK, INF), jnp.float32),
            pltpu.VMEM((TOPK, INF), jnp.float32),
            pltpu.VMEM((TOPK, INF), jnp.float32),
            pltpu.VMEM((INF,), jnp.float32),
            pltpu.SemaphoreType.DMA,
            pltpu.SemaphoreType.DMA,
            pltpu.SemaphoreType.DMA,
            pltpu.SemaphoreType.DMA,
        ],
    )
    eb = sc_fn(fi3.reshape(-1), w3.reshape(-1), embout).reshape(SEQ, INF)

    out = pl.pallas_call(
        _add_body,
        grid=(NTB,),
        in_specs=[
            pl.BlockSpec((TB, INF), lambda tb: (tb, 0)),
            pl.BlockSpec((TB, INF), lambda tb: (tb, 0)),
        ],
        out_specs=pl.BlockSpec((TB, INF), lambda tb: (tb, 0)),
        out_shape=jax.ShapeDtypeStruct((SEQ, INF), jnp.float32),
    )(res, eb)
    return out.reshape(1, SEQ, INF)


# speedup vs baseline: 1.2082x; 1.1332x over previous
import functools

import jax
import jax.numpy as jnp
from jax import lax
from jax.experimental import pallas as pl
from jax.experimental.pallas import tpu as pltpu
from jax.experimental.pallas import tpu_sc as plsc

NHEAD = 8
QDIM = 512
TOPK = 8
NQ = 256
SEQ = 2048
INF = 1024
TB = 256
NTB = SEQ // TB
EMB_ROWS = (NQ - 1) * TOPK + (NQ - 1) + 1
EMB_PAD = 2304

_SQRT_2_OVER_PI = 0.7978845608028654


def _top8_packed(s, nbits):
    rows, n = s.shape
    mask = (1 << nbits) - 1
    iota = jax.lax.broadcasted_iota(jnp.int32, (rows, n), 1)
    si = jax.lax.bitcast_convert_type(s, jnp.int32)
    ki = jnp.bitwise_or(jnp.bitwise_and(si, jnp.int32(~mask)), mask - iota)
    key = jax.lax.bitcast_convert_type(ki, jnp.float32)
    vals, idxs = [], []
    for _ in range(TOPK):
        m = jnp.max(key, axis=-1, keepdims=True)
        key = jnp.where(key == m, -jnp.inf, key)
        mb = jax.lax.bitcast_convert_type(m, jnp.int32)
        vals.append(m)
        idxs.append(mask - jnp.bitwise_and(mb, jnp.int32(mask)))
    return jnp.concatenate(vals, axis=1), jnp.concatenate(idxs, axis=1)


def _nt(a, b):
    return jax.lax.dot_general(a, b, (((1,), (1,)), ((), ())),
                               preferred_element_type=jnp.float32)


def _fused_body(inp_f32, inp_bf, wres, wq, wl, wr, wk, embin,
                res_ref, fi_ref, w_ref):
    h = pl.program_id(1)

    x = _nt(inp_bf[...], wq[0])
    xb = x.astype(jnp.bfloat16)
    sl = _nt(xb, wl[...])
    sr = _nt(xb, wr[...])

    lv, li = _top8_packed(sl, 8)
    rv, ri = _top8_packed(sr, 8)

    lrep = jnp.concatenate(
        [jnp.broadcast_to(lv[:, a:a + 1], (TB, TOPK)) for a in range(TOPK)],
        axis=1)
    rtil = jnp.concatenate([rv] * TOPK, axis=1)
    lirep = jnp.concatenate(
        [jnp.broadcast_to(li[:, a:a + 1], (TB, TOPK)) for a in range(TOPK)],
        axis=1)
    ritil = jnp.concatenate([ri] * TOPK, axis=1)
    payload = jnp.bitwise_or(jnp.left_shift(lirep, 8), ritil)
    ci = jnp.bitwise_or(
        jnp.bitwise_and(jax.lax.bitcast_convert_type(lrep + rtil, jnp.int32),
                        jnp.int32(~0xFFFF)), payload)
    ckey = jax.lax.bitcast_convert_type(ci, jnp.float32)

    dots, fibits = [], []
    for _ in range(TOPK):
        m = jnp.max(ckey, axis=-1, keepdims=True)
        ckey = jnp.where(ckey == m, -jnp.inf, ckey)
        dots.append(m)
        fibits.append(jax.lax.bitcast_convert_type(m, jnp.int32))
    dot = jnp.concatenate(dots, axis=1)
    fib = jnp.concatenate(fibits, axis=1)
    fi = (jnp.bitwise_and(jnp.right_shift(fib, 8), 0xFF) * TOPK
          + jnp.bitwise_and(fib, 0xFF))

    e = jnp.exp(dot - jnp.max(dot, axis=-1, keepdims=True))
    scores = e / jnp.sum(e, axis=-1, keepdims=True)

    proj = _nt(inp_bf[...], wk[0])
    ad = _nt(proj.astype(jnp.bfloat16), embin[...])

    lane = jnp.bitwise_and(fi, 127)
    bsel = jnp.right_shift(fi, 7)
    in_dot = jnp.zeros((TB, TOPK), jnp.float32)
    for j in range(EMB_PAD // 128):
        g = jnp.take_along_axis(ad[:, j * 128:(j + 1) * 128], lane, axis=1,
                                mode='promise_in_bounds')
        in_dot = in_dot + jnp.where(bsel == j, g, 0.0)

    g = 0.5 * in_dot * (1.0 + jnp.tanh(
        _SQRT_2_OVER_PI * (in_dot + 0.044715 * in_dot * in_dot * in_dot)))
    w = scores * g

    cols = jax.lax.broadcasted_iota(jnp.int32, (TB, NHEAD * TOPK), 1)
    base = h * TOPK
    nfi = fi_ref[...]
    nw = w_ref[...]
    for k in range(TOPK):
        hit = cols == base + k
        nfi = jnp.where(hit, fi[:, k:k + 1], nfi)
        nw = jnp.where(hit, w[:, k:k + 1], nw)
    fi_ref[...] = nfi
    w_ref[...] = nw

    @pl.when(h == 0)
    def _():
        res_ref[...] = _nt(inp_f32[...], wres[...])


SC_NC = 2
SC_NS = 16
SC_NW = SC_NC * SC_NS
TOK_PER_TILE = SEQ // SC_NW


ROWS_PER_DMA = 16
NQUART = (NHEAD * TOPK) // ROWS_PER_DMA


def _sc_combine(fi_hbm, w_hbm, emb_hbm, out_hbm,
                fi_v, w_v, rows0, rows1, rows2, rows3, eacc,
                sem0, sem1, sem2, sem3):
    wid = lax.axis_index("s") * SC_NC + lax.axis_index("c")
    nsel = TOK_PER_TILE * NHEAD * TOPK

    pltpu.sync_copy(fi_hbm.at[pl.ds(wid * nsel, nsel)], fi_v)
    pltpu.sync_copy(w_hbm.at[pl.ds(wid * nsel, nsel)],
                    w_v.at[pl.ds(0, nsel)])

    rows = (rows0, rows1, rows2, rows3)
    sems = (sem0, sem1, sem2, sem3)
    tbase = wid * TOK_PER_TILE

    def idx_ref(t, q):
        return fi_v.at[pl.ds(t * (NHEAD * TOPK) + q * ROWS_PER_DMA,
                             ROWS_PER_DMA)]

    for q in range(NQUART):
        pltpu.async_copy(emb_hbm.at[idx_ref(0, q)], rows[q], sems[q])

    gdn = lax.GatherDimensionNumbers(
        offset_dims=(), collapsed_slice_dims=(0,), start_index_map=(0,))

    def token_body(t, carry):
        for q in range(NQUART):
            pltpu.make_async_copy(emb_hbm.at[idx_ref(0, 0)], rows[q],
                                  sems[q]).wait()
            w16 = w_v[pl.ds(t * (NHEAD * TOPK) + q * ROWS_PER_DMA, 16)]
            wk = [lax.gather(w16, jnp.full((16, 1), r, jnp.int32), gdn, (1,),
                             mode=lax.GatherScatterMode.PROMISE_IN_BOUNDS)
                  for r in range(ROWS_PER_DMA)]

            def chunk_body(ci, c2, _q=q, _wk=wk):
                for j in range(4):
                    off = ci * 64 + j * 16
                    p = [_wk[r] * rows[_q][r, pl.ds(off, 16)]
                         for r in range(ROWS_PER_DMA)]
                    s4 = [(p[4 * i] + p[4 * i + 1]) + (p[4 * i + 2]
                                                       + p[4 * i + 3])
                          for i in range(4)]
                    s = (s4[0] + s4[1]) + (s4[2] + s4[3])
                    if _q == 0:
                        eacc[pl.ds(off, 16)] = s
                    else:
                        eacc[pl.ds(off, 16)] = eacc[pl.ds(off, 16)] + s
                return c2

            lax.fori_loop(0, INF // 64, chunk_body, 0)

            @pl.when(t + 1 < TOK_PER_TILE)
            def _():
                pltpu.async_copy(emb_hbm.at[idx_ref(t + 1, q)], rows[q],
                                 sems[q])
        pltpu.sync_copy(eacc, out_hbm.at[pl.ds((tbase + t) * INF, INF)])
        return carry

    lax.fori_loop(0, TOK_PER_TILE, token_body, 0)


def _add_body(res, eb, out_ref):
    out_ref[...] = res[...] + eb[...].astype(jnp.float32)


def kernel(inp, W_res, W_q, W_k, W_left, W_right, emb_in, emb_out):
    inp2d = inp.reshape(SEQ, INF)
    inp_bf = inp2d.astype(jnp.bfloat16)
    wq = W_q.reshape(NHEAD, QDIM, INF).astype(jnp.bfloat16)
    wk = W_k.reshape(NHEAD, INF, INF).astype(jnp.bfloat16)
    wl = W_left.astype(jnp.bfloat16)
    wr = W_right.astype(jnp.bfloat16)
    pad = EMB_PAD - EMB_ROWS
    embin = jnp.pad(emb_in[:EMB_ROWS].astype(jnp.bfloat16), ((0, pad), (0, 0)))

    grid = (NTB, NHEAD)
    res, fi3, w3 = pl.pallas_call(
        _fused_body,
        grid=grid,
        in_specs=[
            pl.BlockSpec((TB, INF), lambda tb, h: (tb, 0)),
            pl.BlockSpec((TB, INF), lambda tb, h: (tb, 0)),
            pl.BlockSpec((INF, INF), lambda tb, h: (0, 0)),
            pl.BlockSpec((1, QDIM, INF), lambda tb, h: (h, 0, 0)),
            pl.BlockSpec((NQ, QDIM), lambda tb, h: (0, 0)),
            pl.BlockSpec((NQ, QDIM), lambda tb, h: (0, 0)),
            pl.BlockSpec((1, INF, INF), lambda tb, h: (h, 0, 0)),
            pl.BlockSpec((EMB_PAD, INF), lambda tb, h: (0, 0)),
        ],
        out_specs=[
            pl.BlockSpec((TB, INF), lambda tb, h: (tb, 0)),
            pl.BlockSpec((TB, NHEAD * TOPK), lambda tb, h: (tb, 0)),
            pl.BlockSpec((TB, NHEAD * TOPK), lambda tb, h: (tb, 0)),
        ],
        out_shape=[
            jax.ShapeDtypeStruct((SEQ, INF), jnp.float32),
            jax.ShapeDtypeStruct((SEQ, NHEAD * TOPK), jnp.int32),
            jax.ShapeDtypeStruct((SEQ, NHEAD * TOPK), jnp.float32),
        ],
        compiler_params=pltpu.CompilerParams(
            dimension_semantics=("arbitrary", "arbitrary")),
    )(inp2d, inp_bf, W_res, wq, wl, wr, wk, embin)

    embout = emb_out[:EMB_PAD]

    sc_fn = pl.kernel(
        _sc_combine,
        mesh=plsc.VectorSubcoreMesh(core_axis_name="c", subcore_axis_name="s"),
        out_type=jax.ShapeDtypeStruct((SEQ * INF,), jnp.float32),
        scratch_types=[
            pltpu.VMEM((NHEAD * TOK_PER_TILE * TOPK,), jnp.int32),
            pltpu.VMEM((NHEAD * TOK_PER_TILE * TOPK + 16,), jnp.float32),
            pltpu.VMEM((ROWS_PER_DMA, INF), jnp.float32),
            pltpu.VMEM((ROWS_PER_DMA, INF), jnp.float32),
            pltpu.VMEM((ROWS_PER_DMA, INF), jnp.float32),
            pltpu.VMEM((ROWS_PER_DMA, INF), jnp.float32),
            pltpu.VMEM((INF,), jnp.float32),
            pltpu.SemaphoreType.DMA,
            pltpu.SemaphoreType.DMA,
            pltpu.SemaphoreType.DMA,
            pltpu.SemaphoreType.DMA,
        ],
    )
    eb = sc_fn(fi3.reshape(-1), w3.reshape(-1), embout).reshape(SEQ, INF)

    out = pl.pallas_call(
        _add_body,
        grid=(NTB,),
        in_specs=[
            pl.BlockSpec((TB, INF), lambda tb: (tb, 0)),
            pl.BlockSpec((TB, INF), lambda tb: (tb, 0)),
        ],
        out_specs=pl.BlockSpec((TB, INF), lambda tb: (tb, 0)),
        out_shape=jax.ShapeDtypeStruct((SEQ, INF), jnp.float32),
    )(res, eb)
    return out.reshape(1, SEQ, INF)


# speedup vs baseline: 1.4385x; 1.1906x over previous
import functools

import jax
import jax.numpy as jnp
from jax import lax
from jax.experimental import pallas as pl
from jax.experimental.pallas import tpu as pltpu
from jax.experimental.pallas import tpu_sc as plsc

NHEAD = 8
QDIM = 512
TOPK = 8
NQ = 256
SEQ = 2048
INF = 1024
TB = 256
NTB = SEQ // TB
EMB_ROWS = (NQ - 1) * TOPK + (NQ - 1) + 1
EMB_PAD = 2304

_SQRT_2_OVER_PI = 0.7978845608028654


def _top8_packed(s, nbits):
    rows, n = s.shape
    mask = (1 << nbits) - 1
    iota = jax.lax.broadcasted_iota(jnp.int32, (rows, n), 1)
    si = jax.lax.bitcast_convert_type(s, jnp.int32)
    ki = jnp.bitwise_or(jnp.bitwise_and(si, jnp.int32(~mask)), mask - iota)
    key = jax.lax.bitcast_convert_type(ki, jnp.float32)
    vals, idxs = [], []
    for _ in range(TOPK):
        m = jnp.max(key, axis=-1, keepdims=True)
        key = jnp.where(key == m, -jnp.inf, key)
        mb = jax.lax.bitcast_convert_type(m, jnp.int32)
        vals.append(m)
        idxs.append(mask - jnp.bitwise_and(mb, jnp.int32(mask)))
    return jnp.concatenate(vals, axis=1), jnp.concatenate(idxs, axis=1)


def _nt(a, b):
    return jax.lax.dot_general(a, b, (((1,), (1,)), ((), ())),
                               preferred_element_type=jnp.float32)


def _fused_body(inp_f32, inp_bf, wres, wq, wl, wr, wk, embin,
                res_ref, fi_ref, w_ref):
    h = pl.program_id(1)

    x = _nt(inp_bf[...], wq[0])
    xb = x.astype(jnp.bfloat16)
    sl = _nt(xb, wl[...])
    sr = _nt(xb, wr[...])

    lv, li = _top8_packed(sl, 8)
    rv, ri = _top8_packed(sr, 8)

    lrep = jnp.concatenate(
        [jnp.broadcast_to(lv[:, a:a + 1], (TB, TOPK)) for a in range(TOPK)],
        axis=1)
    rtil = jnp.concatenate([rv] * TOPK, axis=1)
    lirep = jnp.concatenate(
        [jnp.broadcast_to(li[:, a:a + 1], (TB, TOPK)) for a in range(TOPK)],
        axis=1)
    ritil = jnp.concatenate([ri] * TOPK, axis=1)
    payload = jnp.bitwise_or(jnp.left_shift(lirep, 8), ritil)
    ci = jnp.bitwise_or(
        jnp.bitwise_and(jax.lax.bitcast_convert_type(lrep + rtil, jnp.int32),
                        jnp.int32(~0xFFFF)), payload)
    ckey = jax.lax.bitcast_convert_type(ci, jnp.float32)

    dots, fibits = [], []
    for _ in range(TOPK):
        m = jnp.max(ckey, axis=-1, keepdims=True)
        ckey = jnp.where(ckey == m, -jnp.inf, ckey)
        dots.append(m)
        fibits.append(jax.lax.bitcast_convert_type(m, jnp.int32))
    dot = jnp.concatenate(dots, axis=1)
    fib = jnp.concatenate(fibits, axis=1)
    fi = (jnp.bitwise_and(jnp.right_shift(fib, 8), 0xFF) * TOPK
          + jnp.bitwise_and(fib, 0xFF))

    e = jnp.exp(dot - jnp.max(dot, axis=-1, keepdims=True))
    scores = e / jnp.sum(e, axis=-1, keepdims=True)

    proj = _nt(inp_bf[...], wk[0])
    ad = _nt(proj.astype(jnp.bfloat16), embin[...])

    lane = jnp.bitwise_and(fi, 127)
    bsel = jnp.right_shift(fi, 7)
    in_dot = jnp.zeros((TB, TOPK), jnp.float32)
    for j in range(EMB_PAD // 128):
        g = jnp.take_along_axis(ad[:, j * 128:(j + 1) * 128], lane, axis=1,
                                mode='promise_in_bounds')
        in_dot = in_dot + jnp.where(bsel == j, g, 0.0)

    g = 0.5 * in_dot * (1.0 + jnp.tanh(
        _SQRT_2_OVER_PI * (in_dot + 0.044715 * in_dot * in_dot * in_dot)))
    w = scores * g

    cols = jax.lax.broadcasted_iota(jnp.int32, (TB, NHEAD * TOPK), 1)
    base = h * TOPK
    nfi = fi_ref[...]
    nw = w_ref[...]
    for k in range(TOPK):
        hit = cols == base + k
        nfi = jnp.where(hit, fi[:, k:k + 1], nfi)
        nw = jnp.where(hit, w[:, k:k + 1], nw)
    fi_ref[...] = nfi
    w_ref[...] = nw

    @pl.when(h == 0)
    def _():
        res_ref[...] = _nt(inp_f32[...], wres[...])


SC_NC = 2
SC_NS = 16
SC_NW = SC_NC * SC_NS
TOK_PER_TILE = SEQ // SC_NW


ROWS_PER_DMA = 16
NQUART = (NHEAD * TOPK) // ROWS_PER_DMA


def _sc_combine(fi_hbm, w_hbm, emb_hbm, out_hbm,
                fi_v, w_v, rows0, rows1, rows2, rows3, eacc,
                sem0, sem1, sem2, sem3, tok_per_tile=None):
    TOK_PER_TILE = tok_per_tile
    wid = lax.axis_index("s") * SC_NC + lax.axis_index("c")
    nsel = TOK_PER_TILE * NHEAD * TOPK

    pltpu.sync_copy(fi_hbm.at[pl.ds(wid * nsel, nsel)], fi_v)
    pltpu.sync_copy(w_hbm.at[pl.ds(wid * nsel, nsel)],
                    w_v.at[pl.ds(0, nsel)])

    rows = (rows0, rows1, rows2, rows3)
    sems = (sem0, sem1, sem2, sem3)
    tbase = wid * TOK_PER_TILE

    def idx_ref(t, q):
        return fi_v.at[pl.ds(t * (NHEAD * TOPK) + q * ROWS_PER_DMA,
                             ROWS_PER_DMA)]

    for q in range(NQUART):
        pltpu.async_copy(emb_hbm.at[idx_ref(0, q)], rows[q], sems[q])

    gdn = lax.GatherDimensionNumbers(
        offset_dims=(), collapsed_slice_dims=(0,), start_index_map=(0,))

    def token_body(t, carry):
        for q in range(NQUART):
            pltpu.make_async_copy(emb_hbm.at[idx_ref(0, 0)], rows[q],
                                  sems[q]).wait()
            w16 = w_v[pl.ds(t * (NHEAD * TOPK) + q * ROWS_PER_DMA, 16)]
            wk = [lax.gather(w16, jnp.full((16, 1), r, jnp.int32), gdn, (1,),
                             mode=lax.GatherScatterMode.PROMISE_IN_BOUNDS)
                  for r in range(ROWS_PER_DMA)]

            def chunk_body(ci, c2, _q=q, _wk=wk):
                for j in range(4):
                    off = ci * 64 + j * 16
                    p = [_wk[r] * rows[_q][r, pl.ds(off, 16)]
                         for r in range(ROWS_PER_DMA)]
                    s4 = [(p[4 * i] + p[4 * i + 1]) + (p[4 * i + 2]
                                                       + p[4 * i + 3])
                          for i in range(4)]
                    s = (s4[0] + s4[1]) + (s4[2] + s4[3])
                    if _q == 0:
                        eacc[pl.ds(off, 16)] = s
                    else:
                        eacc[pl.ds(off, 16)] = eacc[pl.ds(off, 16)] + s
                return c2

            lax.fori_loop(0, INF // 64, chunk_body, 0)

            @pl.when(t + 1 < TOK_PER_TILE)
            def _():
                pltpu.async_copy(emb_hbm.at[idx_ref(t + 1, q)], rows[q],
                                 sems[q])
        pltpu.sync_copy(eacc, out_hbm.at[pl.ds((tbase + t) * INF, INF)])
        return carry

    lax.fori_loop(0, TOK_PER_TILE, token_body, 0)


def _add_body(res, eb, out_ref):
    out_ref[...] = res[...] + eb[...].astype(jnp.float32)


def _make_tc(rows):
    ntb = rows // TB
    return pl.pallas_call(
        _fused_body,
        grid=(ntb, NHEAD),
        in_specs=[
            pl.BlockSpec((TB, INF), lambda tb, h: (tb, 0)),
            pl.BlockSpec((TB, INF), lambda tb, h: (tb, 0)),
            pl.BlockSpec((INF, INF), lambda tb, h: (0, 0)),
            pl.BlockSpec((1, QDIM, INF), lambda tb, h: (h, 0, 0)),
            pl.BlockSpec((NQ, QDIM), lambda tb, h: (0, 0)),
            pl.BlockSpec((NQ, QDIM), lambda tb, h: (0, 0)),
            pl.BlockSpec((1, INF, INF), lambda tb, h: (h, 0, 0)),
            pl.BlockSpec((EMB_PAD, INF), lambda tb, h: (0, 0)),
        ],
        out_specs=[
            pl.BlockSpec((TB, INF), lambda tb, h: (tb, 0)),
            pl.BlockSpec((TB, NHEAD * TOPK), lambda tb, h: (tb, 0)),
            pl.BlockSpec((TB, NHEAD * TOPK), lambda tb, h: (tb, 0)),
        ],
        out_shape=[
            jax.ShapeDtypeStruct((rows, INF), jnp.float32),
            jax.ShapeDtypeStruct((rows, NHEAD * TOPK), jnp.int32),
            jax.ShapeDtypeStruct((rows, NHEAD * TOPK), jnp.float32),
        ],
        compiler_params=pltpu.CompilerParams(
            dimension_semantics=("arbitrary", "arbitrary")),
    )


def _make_sc(rows):
    tpt = rows // SC_NW
    return pl.kernel(
        functools.partial(_sc_combine, tok_per_tile=tpt),
        mesh=plsc.VectorSubcoreMesh(core_axis_name="c", subcore_axis_name="s"),
        out_type=jax.ShapeDtypeStruct((rows * INF,), jnp.float32),
        scratch_types=[
            pltpu.VMEM((NHEAD * tpt * TOPK,), jnp.int32),
            pltpu.VMEM((NHEAD * tpt * TOPK + 16,), jnp.float32),
            pltpu.VMEM((ROWS_PER_DMA, INF), jnp.float32),
            pltpu.VMEM((ROWS_PER_DMA, INF), jnp.float32),
            pltpu.VMEM((ROWS_PER_DMA, INF), jnp.float32),
            pltpu.VMEM((ROWS_PER_DMA, INF), jnp.float32),
            pltpu.VMEM((INF,), jnp.float32),
            pltpu.SemaphoreType.DMA,
            pltpu.SemaphoreType.DMA,
            pltpu.SemaphoreType.DMA,
            pltpu.SemaphoreType.DMA,
        ],
    )


NPHASE = 2
PH_ROWS = SEQ // NPHASE


def kernel(inp, W_res, W_q, W_k, W_left, W_right, emb_in, emb_out):
    inp2d = inp.reshape(SEQ, INF)
    inp_bf = inp2d.astype(jnp.bfloat16)
    wq = W_q.reshape(NHEAD, QDIM, INF).astype(jnp.bfloat16)
    wk = W_k.reshape(NHEAD, INF, INF).astype(jnp.bfloat16)
    wl = W_left.astype(jnp.bfloat16)
    wr = W_right.astype(jnp.bfloat16)
    pad = EMB_PAD - EMB_ROWS
    embin = jnp.pad(emb_in[:EMB_ROWS].astype(jnp.bfloat16), ((0, pad), (0, 0)))
    embout = emb_out[:EMB_PAD]

    tc_fn = _make_tc(PH_ROWS)
    sc_fn = _make_sc(PH_ROWS)

    tc_outs = []
    for p in range(NPHASE):
        sl = slice(p * PH_ROWS, (p + 1) * PH_ROWS)
        tc_outs.append(tc_fn(inp2d[sl], inp_bf[sl], W_res, wq, wl, wr, wk,
                             embin))
    ebs = [sc_fn(fi.reshape(-1), w.reshape(-1), embout).reshape(PH_ROWS, INF)
           for (_, fi, w) in tc_outs]

    add_fn = pl.pallas_call(
        _add_body,
        grid=(PH_ROWS // TB,),
        in_specs=[
            pl.BlockSpec((TB, INF), lambda tb: (tb, 0)),
            pl.BlockSpec((TB, INF), lambda tb: (tb, 0)),
        ],
        out_specs=pl.BlockSpec((TB, INF), lambda tb: (tb, 0)),
        out_shape=jax.ShapeDtypeStruct((PH_ROWS, INF), jnp.float32),
    )
    outs = [add_fn(tc_outs[p][0], ebs[p]) for p in range(NPHASE)]
    return jnp.concatenate(outs, axis=0).reshape(1, SEQ, INF)


# speedup vs baseline: 1.5846x; 1.1016x over previous
import functools

import jax
import jax.numpy as jnp
from jax import lax
from jax.experimental import pallas as pl
from jax.experimental.pallas import tpu as pltpu
from jax.experimental.pallas import tpu_sc as plsc

NHEAD = 8
QDIM = 512
TOPK = 8
NQ = 256
SEQ = 2048
INF = 1024
TB = 256
NTB = SEQ // TB
EMB_ROWS = (NQ - 1) * TOPK + (NQ - 1) + 1
EMB_PAD = 2304

_SQRT_2_OVER_PI = 0.7978845608028654


def _top8_packed(s, nbits):
    rows, n = s.shape
    mask = (1 << nbits) - 1
    iota = jax.lax.broadcasted_iota(jnp.int32, (rows, n), 1)
    si = jax.lax.bitcast_convert_type(s, jnp.int32)
    ki = jnp.bitwise_or(jnp.bitwise_and(si, jnp.int32(~mask)), mask - iota)
    key = jax.lax.bitcast_convert_type(ki, jnp.float32)
    vals, idxs = [], []
    for _ in range(TOPK):
        m = jnp.max(key, axis=-1, keepdims=True)
        key = jnp.where(key == m, -jnp.inf, key)
        mb = jax.lax.bitcast_convert_type(m, jnp.int32)
        vals.append(m)
        idxs.append(mask - jnp.bitwise_and(mb, jnp.int32(mask)))
    return jnp.concatenate(vals, axis=1), jnp.concatenate(idxs, axis=1)


def _nt(a, b):
    return jax.lax.dot_general(a, b, (((1,), (1,)), ((), ())),
                               preferred_element_type=jnp.float32)


def _fused_body(inp_f32, inp_bf, wres, wq, wl, wr, wk, embin,
                res_ref, fi_ref, w_ref):
    h = pl.program_id(1)

    x = _nt(inp_bf[...], wq[0])
    xb = x.astype(jnp.bfloat16)
    sl = _nt(xb, wl[...])
    sr = _nt(xb, wr[...])

    lv, li = _top8_packed(sl, 8)
    rv, ri = _top8_packed(sr, 8)

    lrep = jnp.concatenate(
        [jnp.broadcast_to(lv[:, a:a + 1], (TB, TOPK)) for a in range(TOPK)],
        axis=1)
    rtil = jnp.concatenate([rv] * TOPK, axis=1)
    lirep = jnp.concatenate(
        [jnp.broadcast_to(li[:, a:a + 1], (TB, TOPK)) for a in range(TOPK)],
        axis=1)
    ritil = jnp.concatenate([ri] * TOPK, axis=1)
    payload = jnp.bitwise_or(jnp.left_shift(lirep, 8), ritil)
    ci = jnp.bitwise_or(
        jnp.bitwise_and(jax.lax.bitcast_convert_type(lrep + rtil, jnp.int32),
                        jnp.int32(~0xFFFF)), payload)
    ckey = jax.lax.bitcast_convert_type(ci, jnp.float32)

    dots, fibits = [], []
    for _ in range(TOPK):
        m = jnp.max(ckey, axis=-1, keepdims=True)
        ckey = jnp.where(ckey == m, -jnp.inf, ckey)
        dots.append(m)
        fibits.append(jax.lax.bitcast_convert_type(m, jnp.int32))
    dot = jnp.concatenate(dots, axis=1)
    fib = jnp.concatenate(fibits, axis=1)
    fi = (jnp.bitwise_and(jnp.right_shift(fib, 8), 0xFF) * TOPK
          + jnp.bitwise_and(fib, 0xFF))

    e = jnp.exp(dot - jnp.max(dot, axis=-1, keepdims=True))
    scores = e / jnp.sum(e, axis=-1, keepdims=True)

    proj = _nt(inp_bf[...], wk[0])
    ad = _nt(proj.astype(jnp.bfloat16), embin[...])

    lane = jnp.bitwise_and(fi, 127)
    bsel = jnp.right_shift(fi, 7)
    in_dot = jnp.zeros((TB, TOPK), jnp.float32)
    for j in range(EMB_PAD // 128):
        g = jnp.take_along_axis(ad[:, j * 128:(j + 1) * 128], lane, axis=1,
                                mode='promise_in_bounds')
        in_dot = in_dot + jnp.where(bsel == j, g, 0.0)

    g = 0.5 * in_dot * (1.0 + jnp.tanh(
        _SQRT_2_OVER_PI * (in_dot + 0.044715 * in_dot * in_dot * in_dot)))
    w = scores * g

    cols = jax.lax.broadcasted_iota(jnp.int32, (TB, NHEAD * TOPK), 1)
    base = h * TOPK
    nfi = fi_ref[...]
    nw = w_ref[...]
    for k in range(TOPK):
        hit = cols == base + k
        nfi = jnp.where(hit, fi[:, k:k + 1], nfi)
        nw = jnp.where(hit, w[:, k:k + 1], nw)
    fi_ref[...] = nfi
    w_ref[...] = nw

    @pl.when(h == 0)
    def _():
        res_ref[...] = _nt(inp_f32[...], wres[...])


SC_NC = 2
SC_NS = 16
SC_NW = SC_NC * SC_NS
TOK_PER_TILE = SEQ // SC_NW


ROWS_PER_DMA = 16
NQUART = (NHEAD * TOPK) // ROWS_PER_DMA


def _sc_combine(fi_hbm, w_hbm, emb_hbm, out_hbm,
                fi_v, w_v, rows0, rows1, rows2, rows3, eacc,
                sem0, sem1, sem2, sem3, tok_per_tile=None):
    TOK_PER_TILE = tok_per_tile
    wid = lax.axis_index("s") * SC_NC + lax.axis_index("c")
    nsel = TOK_PER_TILE * NHEAD * TOPK

    pltpu.sync_copy(fi_hbm.at[pl.ds(wid * nsel, nsel)], fi_v)
    pltpu.sync_copy(w_hbm.at[pl.ds(wid * nsel, nsel)],
                    w_v.at[pl.ds(0, nsel)])

    rows = (rows0, rows1, rows2, rows3)
    sems = (sem0, sem1, sem2, sem3)
    tbase = wid * TOK_PER_TILE

    def idx_ref(t, q):
        return fi_v.at[pl.ds(t * (NHEAD * TOPK) + q * ROWS_PER_DMA,
                             ROWS_PER_DMA)]

    for q in range(NQUART):
        pltpu.async_copy(emb_hbm.at[idx_ref(0, q)], rows[q], sems[q])

    gdn = lax.GatherDimensionNumbers(
        offset_dims=(), collapsed_slice_dims=(0,), start_index_map=(0,))

    def token_body(t, carry):
        for q in range(NQUART):
            pltpu.make_async_copy(emb_hbm.at[idx_ref(0, 0)], rows[q],
                                  sems[q]).wait()
            w16 = w_v[pl.ds(t * (NHEAD * TOPK) + q * ROWS_PER_DMA, 16)]
            wk = [lax.gather(w16, jnp.full((16, 1), r, jnp.int32), gdn, (1,),
                             mode=lax.GatherScatterMode.PROMISE_IN_BOUNDS)
                  for r in range(ROWS_PER_DMA)]

            def chunk_body(ci, c2, _q=q, _wk=wk):
                for j in range(4):
                    off = ci * 64 + j * 16
                    p = [_wk[r] * rows[_q][r, pl.ds(off, 16)]
                         for r in range(ROWS_PER_DMA)]
                    s4 = [(p[4 * i] + p[4 * i + 1]) + (p[4 * i + 2]
                                                       + p[4 * i + 3])
                          for i in range(4)]
                    s = (s4[0] + s4[1]) + (s4[2] + s4[3])
                    if _q == 0:
                        eacc[pl.ds(off, 16)] = s
                    else:
                        eacc[pl.ds(off, 16)] = eacc[pl.ds(off, 16)] + s
                return c2

            lax.fori_loop(0, INF // 64, chunk_body, 0)

            @pl.when(t + 1 < TOK_PER_TILE)
            def _():
                pltpu.async_copy(emb_hbm.at[idx_ref(t + 1, q)], rows[q],
                                 sems[q])
        pltpu.sync_copy(eacc, out_hbm.at[pl.ds((tbase + t) * INF, INF)])
        return carry

    lax.fori_loop(0, TOK_PER_TILE, token_body, 0)


def _add_body(res, eb, out_ref):
    out_ref[...] = res[...] + eb[...].astype(jnp.float32)


def _make_tc(rows):
    ntb = rows // TB
    return pl.pallas_call(
        _fused_body,
        grid=(ntb, NHEAD),
        in_specs=[
            pl.BlockSpec((TB, INF), lambda tb, h: (tb, 0)),
            pl.BlockSpec((TB, INF), lambda tb, h: (tb, 0)),
            pl.BlockSpec((INF, INF), lambda tb, h: (0, 0)),
            pl.BlockSpec((1, QDIM, INF), lambda tb, h: (h, 0, 0)),
            pl.BlockSpec((NQ, QDIM), lambda tb, h: (0, 0)),
            pl.BlockSpec((NQ, QDIM), lambda tb, h: (0, 0)),
            pl.BlockSpec((1, INF, INF), lambda tb, h: (h, 0, 0)),
            pl.BlockSpec((EMB_PAD, INF), lambda tb, h: (0, 0)),
        ],
        out_specs=[
            pl.BlockSpec((TB, INF), lambda tb, h: (tb, 0)),
            pl.BlockSpec((TB, NHEAD * TOPK), lambda tb, h: (tb, 0)),
            pl.BlockSpec((TB, NHEAD * TOPK), lambda tb, h: (tb, 0)),
        ],
        out_shape=[
            jax.ShapeDtypeStruct((rows, INF), jnp.float32),
            jax.ShapeDtypeStruct((rows, NHEAD * TOPK), jnp.int32),
            jax.ShapeDtypeStruct((rows, NHEAD * TOPK), jnp.float32),
        ],
        compiler_params=pltpu.CompilerParams(
            dimension_semantics=("arbitrary", "arbitrary")),
    )


def _make_sc(rows):
    tpt = rows // SC_NW
    return pl.kernel(
        functools.partial(_sc_combine, tok_per_tile=tpt),
        mesh=plsc.VectorSubcoreMesh(core_axis_name="c", subcore_axis_name="s"),
        out_type=jax.ShapeDtypeStruct((rows * INF,), jnp.float32),
        scratch_types=[
            pltpu.VMEM((NHEAD * tpt * TOPK,), jnp.int32),
            pltpu.VMEM((NHEAD * tpt * TOPK + 16,), jnp.float32),
            pltpu.VMEM((ROWS_PER_DMA, INF), jnp.float32),
            pltpu.VMEM((ROWS_PER_DMA, INF), jnp.float32),
            pltpu.VMEM((ROWS_PER_DMA, INF), jnp.float32),
            pltpu.VMEM((ROWS_PER_DMA, INF), jnp.float32),
            pltpu.VMEM((INF,), jnp.float32),
            pltpu.SemaphoreType.DMA,
            pltpu.SemaphoreType.DMA,
            pltpu.SemaphoreType.DMA,
            pltpu.SemaphoreType.DMA,
        ],
    )


NPHASE = 4
PH_ROWS = SEQ // NPHASE


def kernel(inp, W_res, W_q, W_k, W_left, W_right, emb_in, emb_out):
    inp2d = inp.reshape(SEQ, INF)
    inp_bf = inp2d.astype(jnp.bfloat16)
    wq = W_q.reshape(NHEAD, QDIM, INF).astype(jnp.bfloat16)
    wk = W_k.reshape(NHEAD, INF, INF).astype(jnp.bfloat16)
    wl = W_left.astype(jnp.bfloat16)
    wr = W_right.astype(jnp.bfloat16)
    pad = EMB_PAD - EMB_ROWS
    embin = jnp.pad(emb_in[:EMB_ROWS].astype(jnp.bfloat16), ((0, pad), (0, 0)))
    embout = emb_out[:EMB_PAD]

    tc_fn = _make_tc(PH_ROWS)
    sc_fn = _make_sc(PH_ROWS)

    tc_outs = []
    for p in range(NPHASE):
        sl = slice(p * PH_ROWS, (p + 1) * PH_ROWS)
        tc_outs.append(tc_fn(inp2d[sl], inp_bf[sl], W_res, wq, wl, wr, wk,
                             embin))
    ebs = [sc_fn(fi.reshape(-1), w.reshape(-1), embout).reshape(PH_ROWS, INF)
           for (_, fi, w) in tc_outs]

    add_fn = pl.pallas_call(
        _add_body,
        grid=(PH_ROWS // TB,),
        in_specs=[
            pl.BlockSpec((TB, INF), lambda tb: (tb, 0)),
            pl.BlockSpec((TB, INF), lambda tb: (tb, 0)),
        ],
        out_specs=pl.BlockSpec((TB, INF), lambda tb: (tb, 0)),
        out_shape=jax.ShapeDtypeStruct((PH_ROWS, INF), jnp.float32),
    )
    outs = [add_fn(tc_outs[p][0], ebs[p]) for p in range(NPHASE)]
    return jnp.concatenate(outs, axis=0).reshape(1, SEQ, INF)


# speedup vs baseline: 1.5906x; 1.0038x over previous
import functools

import jax
import jax.numpy as jnp
from jax import lax
from jax.experimental import pallas as pl
from jax.experimental.pallas import tpu as pltpu
from jax.experimental.pallas import tpu_sc as plsc

NHEAD = 8
QDIM = 512
TOPK = 8
NQ = 256
SEQ = 2048
INF = 1024
TB = 256
NTB = SEQ // TB
EMB_ROWS = (NQ - 1) * TOPK + (NQ - 1) + 1
EMB_PAD = 2304

_SQRT_2_OVER_PI = 0.7978845608028654


def _top8_packed(s, nbits):
    rows, n = s.shape
    mask = (1 << nbits) - 1
    iota = jax.lax.broadcasted_iota(jnp.int32, (rows, n), 1)
    si = jax.lax.bitcast_convert_type(s, jnp.int32)
    ki = jnp.bitwise_or(jnp.bitwise_and(si, jnp.int32(~mask)), mask - iota)
    key = jax.lax.bitcast_convert_type(ki, jnp.float32)
    vals, idxs = [], []
    for _ in range(TOPK):
        m = jnp.max(key, axis=-1, keepdims=True)
        key = jnp.where(key == m, -jnp.inf, key)
        mb = jax.lax.bitcast_convert_type(m, jnp.int32)
        vals.append(m)
        idxs.append(mask - jnp.bitwise_and(mb, jnp.int32(mask)))
    return jnp.concatenate(vals, axis=1), jnp.concatenate(idxs, axis=1)


def _nt(a, b):
    return jax.lax.dot_general(a, b, (((1,), (1,)), ((), ())),
                               preferred_element_type=jnp.float32)


def _fused_body(inp_f32, inp_bf, wres, wq, wl, wr, wk, embin,
                res_ref, fi_ref, w_ref):
    h = pl.program_id(1)

    x = _nt(inp_bf[...], wq[0])
    xb = x.astype(jnp.bfloat16)
    sl = _nt(xb, wl[...])
    sr = _nt(xb, wr[...])

    lv, li = _top8_packed(sl, 8)
    rv, ri = _top8_packed(sr, 8)

    lrep = jnp.concatenate(
        [jnp.broadcast_to(lv[:, a:a + 1], (TB, TOPK)) for a in range(TOPK)],
        axis=1)
    rtil = jnp.concatenate([rv] * TOPK, axis=1)
    lirep = jnp.concatenate(
        [jnp.broadcast_to(li[:, a:a + 1], (TB, TOPK)) for a in range(TOPK)],
        axis=1)
    ritil = jnp.concatenate([ri] * TOPK, axis=1)
    payload = jnp.bitwise_or(jnp.left_shift(lirep, 8), ritil)
    ci = jnp.bitwise_or(
        jnp.bitwise_and(jax.lax.bitcast_convert_type(lrep + rtil, jnp.int32),
                        jnp.int32(~0xFFFF)), payload)
    ckey = jax.lax.bitcast_convert_type(ci, jnp.float32)

    dots, fibits = [], []
    for _ in range(TOPK):
        m = jnp.max(ckey, axis=-1, keepdims=True)
        ckey = jnp.where(ckey == m, -jnp.inf, ckey)
        dots.append(m)
        fibits.append(jax.lax.bitcast_convert_type(m, jnp.int32))
    dot = jnp.concatenate(dots, axis=1)
    fib = jnp.concatenate(fibits, axis=1)
    fi = (jnp.bitwise_and(jnp.right_shift(fib, 8), 0xFF) * TOPK
          + jnp.bitwise_and(fib, 0xFF))

    e = jnp.exp(dot - jnp.max(dot, axis=-1, keepdims=True))
    scores = e / jnp.sum(e, axis=-1, keepdims=True)

    proj = _nt(inp_bf[...], wk[0])
    ad = _nt(proj.astype(jnp.bfloat16), embin[...])

    lane = jnp.bitwise_and(fi, 127)
    bsel = jnp.right_shift(fi, 7)
    in_dot = jnp.zeros((TB, TOPK), jnp.float32)
    for j in range(EMB_PAD // 128):
        g = jnp.take_along_axis(ad[:, j * 128:(j + 1) * 128], lane, axis=1,
                                mode='promise_in_bounds')
        in_dot = in_dot + jnp.where(bsel == j, g, 0.0)

    g = 0.5 * in_dot * (1.0 + jnp.tanh(
        _SQRT_2_OVER_PI * (in_dot + 0.044715 * in_dot * in_dot * in_dot)))
    w = scores * g

    cols = jax.lax.broadcasted_iota(jnp.int32, (TB, NHEAD * TOPK), 1)
    base = h * TOPK
    nfi = fi_ref[...]
    nw = w_ref[...]
    for k in range(TOPK):
        hit = cols == base + k
        nfi = jnp.where(hit, fi[:, k:k + 1], nfi)
        nw = jnp.where(hit, w[:, k:k + 1], nw)
    fi_ref[...] = nfi
    w_ref[...] = nw

    @pl.when(h == 0)
    def _():
        res_ref[...] = _nt(inp_f32[...], wres[...])


SC_NC = 2
SC_NS = 16
SC_NW = SC_NC * SC_NS
TOK_PER_TILE = SEQ // SC_NW


ROWS_PER_DMA = 16
NQUART = (NHEAD * TOPK) // ROWS_PER_DMA


def _sc_combine(fi_hbm, w_hbm, emb_hbm, out_hbm,
                fi_v, w_v, rows0, rows1, rows2, rows3, eacc,
                sem0, sem1, sem2, sem3, tok_per_tile=None):
    TOK_PER_TILE = tok_per_tile
    wid = lax.axis_index("s") * SC_NC + lax.axis_index("c")
    nsel = TOK_PER_TILE * NHEAD * TOPK

    pltpu.sync_copy(fi_hbm.at[pl.ds(wid * nsel, nsel)], fi_v)
    pltpu.sync_copy(w_hbm.at[pl.ds(wid * nsel, nsel)],
                    w_v.at[pl.ds(0, nsel)])

    rows = (rows0, rows1, rows2, rows3)
    sems = (sem0, sem1, sem2, sem3)
    tbase = wid * TOK_PER_TILE

    def idx_ref(t, q):
        return fi_v.at[pl.ds(t * (NHEAD * TOPK) + q * ROWS_PER_DMA,
                             ROWS_PER_DMA)]

    for q in range(NQUART):
        pltpu.async_copy(emb_hbm.at[idx_ref(0, q)], rows[q], sems[q])

    gdn = lax.GatherDimensionNumbers(
        offset_dims=(), collapsed_slice_dims=(0,), start_index_map=(0,))

    def token_body(t, carry):
        for q in range(NQUART):
            pltpu.make_async_copy(emb_hbm.at[idx_ref(0, 0)], rows[q],
                                  sems[q]).wait()
            w16 = w_v[pl.ds(t * (NHEAD * TOPK) + q * ROWS_PER_DMA, 16)]
            wk = [lax.gather(w16, jnp.full((16, 1), r, jnp.int32), gdn, (1,),
                             mode=lax.GatherScatterMode.PROMISE_IN_BOUNDS)
                  for r in range(ROWS_PER_DMA)]

            def chunk_body(ci, c2, _q=q, _wk=wk):
                for j in range(4):
                    off = ci * 64 + j * 16
                    p = [_wk[r] * rows[_q][r, pl.ds(off, 16)]
                         for r in range(ROWS_PER_DMA)]
                    s4 = [(p[4 * i] + p[4 * i + 1]) + (p[4 * i + 2]
                                                       + p[4 * i + 3])
                          for i in range(4)]
                    s = (s4[0] + s4[1]) + (s4[2] + s4[3])
                    if _q == 0:
                        eacc[pl.ds(off, 16)] = s
                    else:
                        eacc[pl.ds(off, 16)] = eacc[pl.ds(off, 16)] + s
                return c2

            lax.fori_loop(0, INF // 64, chunk_body, 0)

            @pl.when(t + 1 < TOK_PER_TILE)
            def _():
                pltpu.async_copy(emb_hbm.at[idx_ref(t + 1, q)], rows[q],
                                 sems[q])
        pltpu.sync_copy(eacc, out_hbm.at[pl.ds((tbase + t) * INF, INF)])
        return carry

    lax.fori_loop(0, TOK_PER_TILE, token_body, 0)


def _add_body(res, eb, out_ref):
    out_ref[...] = res[...] + eb[...].astype(jnp.float32)


def _make_tc(rows):
    ntb = rows // TB
    return pl.pallas_call(
        _fused_body,
        grid=(ntb, NHEAD),
        in_specs=[
            pl.BlockSpec((TB, INF), lambda tb, h: (tb, 0)),
            pl.BlockSpec((TB, INF), lambda tb, h: (tb, 0)),
            pl.BlockSpec((INF, INF), lambda tb, h: (0, 0)),
            pl.BlockSpec((1, QDIM, INF), lambda tb, h: (h, 0, 0)),
            pl.BlockSpec((NQ, QDIM), lambda tb, h: (0, 0)),
            pl.BlockSpec((NQ, QDIM), lambda tb, h: (0, 0)),
            pl.BlockSpec((1, INF, INF), lambda tb, h: (h, 0, 0)),
            pl.BlockSpec((EMB_PAD, INF), lambda tb, h: (0, 0)),
        ],
        out_specs=[
            pl.BlockSpec((TB, INF), lambda tb, h: (tb, 0)),
            pl.BlockSpec((TB, NHEAD * TOPK), lambda tb, h: (tb, 0)),
            pl.BlockSpec((TB, NHEAD * TOPK), lambda tb, h: (tb, 0)),
        ],
        out_shape=[
            jax.ShapeDtypeStruct((rows, INF), jnp.float32),
            jax.ShapeDtypeStruct((rows, NHEAD * TOPK), jnp.int32),
            jax.ShapeDtypeStruct((rows, NHEAD * TOPK), jnp.float32),
        ],
        compiler_params=pltpu.CompilerParams(
            dimension_semantics=("arbitrary", "arbitrary")),
    )


def _make_sc(rows):
    tpt = rows // SC_NW
    return pl.kernel(
        functools.partial(_sc_combine, tok_per_tile=tpt),
        mesh=plsc.VectorSubcoreMesh(core_axis_name="c", subcore_axis_name="s"),
        out_type=jax.ShapeDtypeStruct((rows * INF,), jnp.float32),
        scratch_types=[
            pltpu.VMEM((NHEAD * tpt * TOPK,), jnp.int32),
            pltpu.VMEM((NHEAD * tpt * TOPK + 16,), jnp.float32),
            pltpu.VMEM((ROWS_PER_DMA, INF), jnp.float32),
            pltpu.VMEM((ROWS_PER_DMA, INF), jnp.float32),
            pltpu.VMEM((ROWS_PER_DMA, INF), jnp.float32),
            pltpu.VMEM((ROWS_PER_DMA, INF), jnp.float32),
            pltpu.VMEM((INF,), jnp.float32),
            pltpu.SemaphoreType.DMA,
            pltpu.SemaphoreType.DMA,
            pltpu.SemaphoreType.DMA,
            pltpu.SemaphoreType.DMA,
        ],
    )


NPHASE = 8
PH_ROWS = SEQ // NPHASE


def kernel(inp, W_res, W_q, W_k, W_left, W_right, emb_in, emb_out):
    inp2d = inp.reshape(SEQ, INF)
    inp_bf = inp2d.astype(jnp.bfloat16)
    wq = W_q.reshape(NHEAD, QDIM, INF).astype(jnp.bfloat16)
    wk = W_k.reshape(NHEAD, INF, INF).astype(jnp.bfloat16)
    wl = W_left.astype(jnp.bfloat16)
    wr = W_right.astype(jnp.bfloat16)
    pad = EMB_PAD - EMB_ROWS
    embin = jnp.pad(emb_in[:EMB_ROWS].astype(jnp.bfloat16), ((0, pad), (0, 0)))
    embout = emb_out[:EMB_PAD]

    tc_fn = _make_tc(PH_ROWS)
    sc_fn = _make_sc(PH_ROWS)

    tc_outs = []
    for p in range(NPHASE):
        sl = slice(p * PH_ROWS, (p + 1) * PH_ROWS)
        tc_outs.append(tc_fn(inp2d[sl], inp_bf[sl], W_res, wq, wl, wr, wk,
                             embin))
    ebs = [sc_fn(fi.reshape(-1), w.reshape(-1), embout).reshape(PH_ROWS, INF)
           for (_, fi, w) in tc_outs]

    add_fn = pl.pallas_call(
        _add_body,
        grid=(PH_ROWS // TB,),
        in_specs=[
            pl.BlockSpec((TB, INF), lambda tb: (tb, 0)),
            pl.BlockSpec((TB, INF), lambda tb: (tb, 0)),
        ],
        out_specs=pl.BlockSpec((TB, INF), lambda tb: (tb, 0)),
        out_shape=jax.ShapeDtypeStruct((PH_ROWS, INF), jnp.float32),
    )
    outs = [add_fn(tc_outs[p][0], ebs[p]) for p in range(NPHASE)]
    return jnp.concatenate(outs, axis=0).reshape(1, SEQ, INF)
